# baseline (speedup 1.0000x reference)
import functools

import jax
import jax.numpy as jnp
from jax import lax
from jax.experimental import pallas as pl
from jax.experimental.pallas import tpu as pltpu
from jax.experimental.pallas import tpu_sc as plsc

_NC = 2
_NS = 16
_NW = _NC * _NS
_L = 128


def _gather_rows(table, idx3, n_rows):
  _, d = table.shape
  quads = n_rows // 512
  nt = quads // _NW
  mesh = plsc.VectorSubcoreMesh(core_axis_name="c", subcore_axis_name="s")

  @functools.partial(
      pl.kernel,
      mesh=mesh,
      out_type=jax.ShapeDtypeStruct((n_rows, d), table.dtype),
      scratch_types=[
          pltpu.VMEM((4, _L), jnp.int32),
          pltpu.VMEM((512, d), table.dtype),
          pltpu.SemaphoreType.DMA,
      ],
  )
  def gk(table_hbm, idx_hbm, out_hbm, idx_v, rows_v, sem):
    wid = lax.axis_index("s") * _NC + lax.axis_index("c")

    def body(u, carry):
      qi = wid + u * _NW
      base = qi * 512
      pltpu.sync_copy(idx_hbm.at[qi], idx_v)
      cps = [
          pltpu.async_copy(table_hbm.at[idx_v.at[j]],
                           rows_v.at[pl.ds(j * _L, _L)], sem)
          for j in range(4)
      ]
      for cp in cps:
        cp.wait()
      pltpu.sync_copy(rows_v, out_hbm.at[pl.ds(base, 512)])
      return carry

    lax.fori_loop(0, nt, body, 0)

  return gk(table, idx3)


def _scatter_add(msg, idx3, n_pad):
  n_msg, d = msg.shape
  pairs = n_msg // 256
  nt = pairs // _NW
  rps = n_pad // _NS
  mesh = plsc.VectorSubcoreMesh(core_axis_name="c", subcore_axis_name="s")

  @functools.partial(
      pl.kernel,
      mesh=mesh,
      out_type=jax.ShapeDtypeStruct((_NC, n_pad, d), msg.dtype),
      scratch_types=[
          pltpu.VMEM((2, _L), jnp.int32),
          pltpu.VMEM((_L, d), msg.dtype),
          pltpu.VMEM((_L, d), msg.dtype),
          pltpu.VMEM_SHARED((n_pad, d), msg.dtype),
          pltpu.SemaphoreType.DMA,
          pltpu.SemaphoreType.DMA,
          pltpu.SemaphoreType.DMA,
          pltpu.SemaphoreType.DMA,
      ],
  )
  def sk(m_hbm, idx_hbm, zeros_hbm, out_hbm, idx_v, buf_a, buf_b, agg_sh,
         sem_la, sem_lb, sem_sa, sem_sb):
    cid = lax.axis_index("c")
    sid = lax.axis_index("s")
    wid = sid * _NC + cid
    pltpu.sync_copy(zeros_hbm.at[pl.ds(sid * rps, rps)],
                    agg_sh.at[pl.ds(sid * rps, rps)])
    plsc.subcore_barrier()

    def body(u, carry):
      pi = wid + u * _NW
      base = pi * 256
      pltpu.sync_copy(idx_hbm.at[pi], idx_v)
      la = pltpu.async_copy(m_hbm.at[pl.ds(base, _L)], buf_a, sem_la)
      lb = pltpu.async_copy(m_hbm.at[pl.ds(base + _L, _L)], buf_b, sem_lb)
      la.wait()
      sa = pltpu.async_copy(buf_a, agg_sh.at[idx_v.at[0]], sem_sa, add=True)
      lb.wait()
      sb = pltpu.async_copy(buf_b, agg_sh.at[idx_v.at[1]], sem_sb, add=True)
      sa.wait()
      sb.wait()
      return carry

    lax.fori_loop(0, nt, body, 0)
    plsc.subcore_barrier()
    pltpu.sync_copy(agg_sh.at[pl.ds(sid * rps, rps)],
                    out_hbm.at[cid, pl.ds(sid * rps, rps)])

  return sk(msg, idx3, jnp.zeros((n_pad, d), msg.dtype))


def _edge_d2(posx, posy, posz, idx3, n_edges, chunk):
  n = posx.shape[0]
  k = chunk // _L
  n_chunks = n_edges // chunk
  mesh = plsc.VectorSubcoreMesh(core_axis_name="c", subcore_axis_name="s")

  @functools.partial(
      pl.kernel,
      mesh=mesh,
      out_type=jax.ShapeDtypeStruct((n_edges,), jnp.float32),
      compiler_params=pltpu.CompilerParams(needs_layout_passes=False),
      scratch_types=[
          pltpu.VMEM((n,), jnp.float32),
          pltpu.VMEM((n,), jnp.float32),
          pltpu.VMEM((n,), jnp.float32),
          pltpu.VMEM((k, _L), jnp.int32),
          pltpu.VMEM((k, _L), jnp.int32),
          pltpu.VMEM((chunk,), jnp.float32),
      ],
  )
  def dk(px_hbm, py_hbm, pz_hbm, idx_hbm, out_hbm, px, py, pz, ir_v, ic_v,
         d2_v):
    wid = lax.axis_index("s") * _NC + lax.axis_index("c")
    pltpu.sync_copy(px_hbm, px)
    pltpu.sync_copy(py_hbm, py)
    pltpu.sync_copy(pz_hbm, pz)
    nt = (n_chunks - wid + _NW - 1) // _NW

    def body(t, carry):
      g = wid + t * _NW
      pltpu.sync_copy(idx_hbm.at[g], ir_v)
      pltpu.sync_copy(idx_hbm.at[g + n_chunks], ic_v)
      for j in range(k):
        def vec(v, c2):
          ir = ir_v[j, pl.ds(v * 16, 16)]
          ic = ic_v[j, pl.ds(v * 16, 16)]
          dx = plsc.load_gather(px, [ir]) - plsc.load_gather(px, [ic])
          dy = plsc.load_gather(py, [ir]) - plsc.load_gather(py, [ic])
          dz = plsc.load_gather(pz, [ir]) - plsc.load_gather(pz, [ic])
          d2_v[pl.ds(j * _L + v * 16, 16)] = dx * dx + dy * dy + dz * dz
          return c2

        lax.fori_loop(0, _L // 16, vec, 0)
      pltpu.sync_copy(d2_v, out_hbm.at[pl.ds(g * chunk, chunk)])
      return carry

    lax.fori_loop(0, nt, body, 0)

  return dk(posx, posy, posz, idx3)


def _edge_mlp(hcat, d2, w1a, w1b, w1d, b1, w2, b2, n_out, col_off, be):
  d = hcat.shape[1]
  nbe = n_out // be

  def body(hr, hc, d2_r, w1a_r, w1b_r, w1d_r, b1_r, w2_r, b2_r, out):
    dist = jnp.sqrt(d2_r[...] + 1e-8)
    t = (jnp.dot(hr[...], w1a_r[...], preferred_element_type=jnp.float32)
         + jnp.dot(hc[...], w1b_r[...], preferred_element_type=jnp.float32)
         + dist * w1d_r[...] + b1_r[...])
    t = t * jax.nn.sigmoid(t)
    out[...] = jnp.dot(t, w2_r[...],
                       preferred_element_type=jnp.float32) + b2_r[...]

  return pl.pallas_call(
      body,
      grid=(nbe,),
      in_specs=[
          pl.BlockSpec((be, d), lambda i: (i, 0)),
          pl.BlockSpec((be, d), lambda i: (i + col_off, 0)),
          pl.BlockSpec((be, 1), lambda i: (i, 0)),
          pl.BlockSpec((d, d), lambda i: (0, 0)),
          pl.BlockSpec((d, d), lambda i: (0, 0)),
          pl.BlockSpec((1, d), lambda i: (0, 0)),
          pl.BlockSpec((1, d), lambda i: (0, 0)),
          pl.BlockSpec((d, d), lambda i: (0, 0)),
          pl.BlockSpec((1, d), lambda i: (0, 0)),
      ],
      out_specs=pl.BlockSpec((be, d), lambda i: (i, 0)),
      out_shape=jax.ShapeDtypeStruct((n_out, d), jnp.float32),
  )(hcat, hcat, d2, w1a, w1b, w1d, b1, w2, b2)


def _node_update(h, agg2, u1a, u1b, ub1, u2, ub2, ln_g, ln_b, bn):
  n, d = h.shape
  nbn = pl.cdiv(n, bn)

  def body(h_r, a_r, u1a_r, u1b_r, ub1_r, u2_r, ub2_r, g_r, b_r, out):
    hv = h_r[...]
    a = a_r[0] + a_r[1]
    t = (jnp.dot(hv, u1a_r[...], preferred_element_type=jnp.float32)
         + jnp.dot(a, u1b_r[...], preferred_element_type=jnp.float32)
         + ub1_r[...])
    t = t * jax.nn.sigmoid(t)
    u = jnp.dot(t, u2_r[...], preferred_element_type=jnp.float32) + ub2_r[...]
    r = hv + u
    mu = jnp.mean(r, axis=-1, keepdims=True)
    var = jnp.mean((r - mu) ** 2, axis=-1, keepdims=True)
    out[...] = (r - mu) / jnp.sqrt(var + 1e-5) * g_r[...] + b_r[...]

  return pl.pallas_call(
      body,
      grid=(nbn,),
      in_specs=[
          pl.BlockSpec((bn, d), lambda i: (i, 0)),
          pl.BlockSpec((2, bn, d), lambda i: (0, i, 0)),
          pl.BlockSpec((d, d), lambda i: (0, 0)),
          pl.BlockSpec((d, d), lambda i: (0, 0)),
          pl.BlockSpec((1, d), lambda i: (0, 0)),
          pl.BlockSpec((d, d), lambda i: (0, 0)),
          pl.BlockSpec((1, d), lambda i: (0, 0)),
          pl.BlockSpec((1, d), lambda i: (0, 0)),
          pl.BlockSpec((1, d), lambda i: (0, 0)),
      ],
      out_specs=pl.BlockSpec((bn, d), lambda i: (i, 0)),
      out_shape=jax.ShapeDtypeStruct((n, d), jnp.float32),
  )(h, agg2, u1a, u1b, ub1, u2, ub2, ln_g, ln_b)


def _mean_rows(h, bn):
  n, d = h.shape
  nbn = pl.cdiv(n, bn)

  def body(h_r, out):
    i = pl.program_id(0)

    @pl.when(i == 0)
    def _():
      out[...] = jnp.zeros_like(out)

    rows = i * bn + lax.broadcasted_iota(jnp.int32, (bn, 1), 0)
    x = jnp.where(rows < n, h_r[...], 0.0)
    out[...] += jnp.sum(x, axis=0, keepdims=True) * (1.0 / n)

  return pl.pallas_call(
      body,
      grid=(nbn,),
      in_specs=[pl.BlockSpec((bn, d), lambda i: (i, 0))],
      out_specs=pl.BlockSpec((1, d), lambda i: (0, 0)),
      out_shape=jax.ShapeDtypeStruct((1, d), jnp.float32),
  )(h)


def kernel(z, pos, edge_index, atom_embed, layers):
  n, d = pos.shape[0], atom_embed.shape[1]
  e = edge_index.shape[1]
  row = edge_index[0].astype(jnp.int32)
  col = edge_index[1].astype(jnp.int32)
  npad = ((n + 2047) // 2048) * 2048

  g_rows = ((2 * e + 16383) // 16384) * 16384
  idxcat = jnp.concatenate(
      [row, col, jnp.zeros((g_rows - 2 * e,), jnp.int32)]).reshape(-1, 4, _L)

  e_pad = ((e + 8191) // 8192) * 8192
  row3 = jnp.concatenate(
      [row, jnp.full((e_pad - e,), npad - 1, jnp.int32)]).reshape(-1, 2, _L)

  posf = pos.astype(jnp.float32)
  z_rows = ((n + 16383) // 16384) * 16384
  z3 = jnp.pad(z.astype(jnp.int32), (0, z_rows - n)).reshape(-1, 4, _L)
  h = _gather_rows(atom_embed.astype(jnp.float32), z3, z_rows)[:n]
  d2 = _edge_d2(posf[:, 0], posf[:, 1], posf[:, 2], idxcat, e, 512)
  d2 = jnp.pad(d2, (0, e_pad - e)).reshape(e_pad, 1)

  for lp in layers:
    w1 = lp['msg_w1']
    hcat = _gather_rows(h, idxcat, g_rows)
    m = _edge_mlp(hcat, d2,
                  w1[:d], w1[d:2 * d], w1[2 * d:].reshape(1, d),
                  lp['msg_b1'].reshape(1, d), lp['msg_w2'],
                  lp['msg_b2'].reshape(1, d), e_pad, e // 640, 640)
    agg2 = _scatter_add(m, row3, npad)
    u1 = lp['upd_w1']
    h = _node_update(h, agg2, u1[:d], u1[d:],
                     lp['upd_b1'].reshape(1, d), lp['upd_w2'],
                     lp['upd_b2'].reshape(1, d),
                     lp['ln_g'].reshape(1, d), lp['ln_b'].reshape(1, d), 512)

  return _mean_rows(h, 512)


# speedup vs baseline: 1.5510x; 1.5510x over previous
import functools

import jax
import jax.numpy as jnp
from jax import lax
from jax.experimental import pallas as pl
from jax.experimental.pallas import tpu as pltpu
from jax.experimental.pallas import tpu_sc as plsc

_NC = 2
_NS = 16
_NW = _NC * _NS
_L = 128


def _gather_rows(table, idx3, n_rows):
  _, d = table.shape
  quads = n_rows // 512
  nt = quads // _NW
  mesh = plsc.VectorSubcoreMesh(core_axis_name="c", subcore_axis_name="s")

  @functools.partial(
      pl.kernel,
      mesh=mesh,
      out_type=jax.ShapeDtypeStruct((n_rows, d), table.dtype),
      scratch_types=[
          pltpu.VMEM((4, _L), jnp.int32),
          pltpu.VMEM((512, d), table.dtype),
          pltpu.SemaphoreType.DMA,
      ],
  )
  def gk(table_hbm, idx_hbm, out_hbm, idx_v, rows_v, sem):
    wid = lax.axis_index("s") * _NC + lax.axis_index("c")

    def body(u, carry):
      qi = wid + u * _NW
      base = qi * 512
      pltpu.sync_copy(idx_hbm.at[qi], idx_v)
      cps = [
          pltpu.async_copy(table_hbm.at[idx_v.at[j]],
                           rows_v.at[pl.ds(j * _L, _L)], sem)
          for j in range(4)
      ]
      for cp in cps:
        cp.wait()
      pltpu.sync_copy(rows_v, out_hbm.at[pl.ds(base, 512)])
      return carry

    lax.fori_loop(0, nt, body, 0)

  return gk(table, idx3)


def _scatter_add(msg, idx3, n_pad):
  n_msg, d = msg.shape
  pairs = n_msg // 256
  nt = pairs // _NW
  rps = n_pad // _NS
  mesh = plsc.VectorSubcoreMesh(core_axis_name="c", subcore_axis_name="s")

  @functools.partial(
      pl.kernel,
      mesh=mesh,
      out_type=jax.ShapeDtypeStruct((_NC, n_pad, d), msg.dtype),
      scratch_types=[
          pltpu.VMEM((2, _L), jnp.int32),
          pltpu.VMEM((_L, d), msg.dtype),
          pltpu.VMEM((_L, d), msg.dtype),
          pltpu.VMEM_SHARED((n_pad, d), msg.dtype),
          pltpu.SemaphoreType.DMA,
          pltpu.SemaphoreType.DMA,
          pltpu.SemaphoreType.DMA,
          pltpu.SemaphoreType.DMA,
      ],
  )
  def sk(m_hbm, idx_hbm, zeros_hbm, out_hbm, idx_v, buf_a, buf_b, agg_sh,
         sem_la, sem_lb, sem_sa, sem_sb):
    cid = lax.axis_index("c")
    sid = lax.axis_index("s")
    wid = sid * _NC + cid
    pltpu.sync_copy(zeros_hbm.at[pl.ds(sid * rps, rps)],
                    agg_sh.at[pl.ds(sid * rps, rps)])
    plsc.subcore_barrier()

    def body(u, carry):
      pi = wid + u * _NW
      base = pi * 256
      pltpu.sync_copy(idx_hbm.at[pi], idx_v)
      la = pltpu.async_copy(m_hbm.at[pl.ds(base, _L)], buf_a, sem_la)
      lb = pltpu.async_copy(m_hbm.at[pl.ds(base + _L, _L)], buf_b, sem_lb)
      la.wait()
      sa = pltpu.async_copy(buf_a, agg_sh.at[idx_v.at[0]], sem_sa, add=True)
      lb.wait()
      sb = pltpu.async_copy(buf_b, agg_sh.at[idx_v.at[1]], sem_sb, add=True)
      sa.wait()
      sb.wait()
      return carry

    lax.fori_loop(0, nt, body, 0)
    plsc.subcore_barrier()
    pltpu.sync_copy(agg_sh.at[pl.ds(sid * rps, rps)],
                    out_hbm.at[cid, pl.ds(sid * rps, rps)])

  return sk(msg, idx3, jnp.zeros((n_pad, d), msg.dtype))


def _edge_d2(posx, posy, posz, idx3, n_edges, chunk):
  n = posx.shape[0]
  k = chunk // _L
  n_chunks = n_edges // chunk
  mesh = plsc.VectorSubcoreMesh(core_axis_name="c", subcore_axis_name="s")

  @functools.partial(
      pl.kernel,
      mesh=mesh,
      out_type=jax.ShapeDtypeStruct((n_edges,), jnp.float32),
      compiler_params=pltpu.CompilerParams(needs_layout_passes=False),
      scratch_types=[
          pltpu.VMEM((n,), jnp.float32),
          pltpu.VMEM((n,), jnp.float32),
          pltpu.VMEM((n,), jnp.float32),
          pltpu.VMEM((k, _L), jnp.int32),
          pltpu.VMEM((k, _L), jnp.int32),
          pltpu.VMEM((chunk,), jnp.float32),
      ],
  )
  def dk(px_hbm, py_hbm, pz_hbm, idx_hbm, out_hbm, px, py, pz, ir_v, ic_v,
         d2_v):
    wid = lax.axis_index("s") * _NC + lax.axis_index("c")
    pltpu.sync_copy(px_hbm, px)
    pltpu.sync_copy(py_hbm, py)
    pltpu.sync_copy(pz_hbm, pz)
    nt = (n_chunks - wid + _NW - 1) // _NW

    def body(t, carry):
      g = wid + t * _NW
      pltpu.sync_copy(idx_hbm.at[g], ir_v)
      pltpu.sync_copy(idx_hbm.at[g + n_chunks], ic_v)
      for j in range(k):
        def vec(v, c2):
          ir = ir_v[j, pl.ds(v * 16, 16)]
          ic = ic_v[j, pl.ds(v * 16, 16)]
          dx = plsc.load_gather(px, [ir]) - plsc.load_gather(px, [ic])
          dy = plsc.load_gather(py, [ir]) - plsc.load_gather(py, [ic])
          dz = plsc.load_gather(pz, [ir]) - plsc.load_gather(pz, [ic])
          d2_v[pl.ds(j * _L + v * 16, 16)] = dx * dx + dy * dy + dz * dz
          return c2

        lax.fori_loop(0, _L // 16, vec, 0)
      pltpu.sync_copy(d2_v, out_hbm.at[pl.ds(g * chunk, chunk)])
      return carry

    lax.fori_loop(0, nt, body, 0)

  return dk(posx, posy, posz, idx3)


def _edge_mlp(hcat, d2, w1a, w1b, w1d, b1, w2, b2, n_out, col_off, be):
  d = hcat.shape[1]
  nbe = n_out // be

  def body(hr, hc, d2_r, w1a_r, w1b_r, w1d_r, b1_r, w2_r, b2_r, out):
    dist = jnp.sqrt(d2_r[...] + 1e-8)
    t = (jnp.dot(hr[...], w1a_r[...], preferred_element_type=jnp.float32)
         + jnp.dot(hc[...], w1b_r[...], preferred_element_type=jnp.float32)
         + dist * w1d_r[...] + b1_r[...])
    t = t * jax.nn.sigmoid(t)
    out[...] = jnp.dot(t, w2_r[...],
                       preferred_element_type=jnp.float32) + b2_r[...]

  return pl.pallas_call(
      body,
      grid=(nbe,),
      in_specs=[
          pl.BlockSpec((be, d), lambda i: (i, 0)),
          pl.BlockSpec((be, d), lambda i: (i + col_off, 0)),
          pl.BlockSpec((be, 1), lambda i: (i, 0)),
          pl.BlockSpec((d, d), lambda i: (0, 0)),
          pl.BlockSpec((d, d), lambda i: (0, 0)),
          pl.BlockSpec((1, d), lambda i: (0, 0)),
          pl.BlockSpec((1, d), lambda i: (0, 0)),
          pl.BlockSpec((d, d), lambda i: (0, 0)),
          pl.BlockSpec((1, d), lambda i: (0, 0)),
      ],
      out_specs=pl.BlockSpec((be, d), lambda i: (i, 0)),
      out_shape=jax.ShapeDtypeStruct((n_out, d), jnp.float32),
  )(hcat, hcat, d2, w1a, w1b, w1d, b1, w2, b2)


def _node_update(h, agg2, u1a, u1b, ub1, u2, ub2, ln_g, ln_b, bn):
  n, d = h.shape
  nbn = pl.cdiv(n, bn)

  def body(h_r, a_r, u1a_r, u1b_r, ub1_r, u2_r, ub2_r, g_r, b_r, out):
    hv = h_r[...]
    a = a_r[0] + a_r[1]
    t = (jnp.dot(hv, u1a_r[...], preferred_element_type=jnp.float32)
         + jnp.dot(a, u1b_r[...], preferred_element_type=jnp.float32)
         + ub1_r[...])
    t = t * jax.nn.sigmoid(t)
    u = jnp.dot(t, u2_r[...], preferred_element_type=jnp.float32) + ub2_r[...]
    r = hv + u
    mu = jnp.mean(r, axis=-1, keepdims=True)
    var = jnp.mean((r - mu) ** 2, axis=-1, keepdims=True)
    out[...] = (r - mu) / jnp.sqrt(var + 1e-5) * g_r[...] + b_r[...]

  return pl.pallas_call(
      body,
      grid=(nbn,),
      in_specs=[
          pl.BlockSpec((bn, d), lambda i: (i, 0)),
          pl.BlockSpec((2, bn, d), lambda i: (0, i, 0)),
          pl.BlockSpec((d, d), lambda i: (0, 0)),
          pl.BlockSpec((d, d), lambda i: (0, 0)),
          pl.BlockSpec((1, d), lambda i: (0, 0)),
          pl.BlockSpec((d, d), lambda i: (0, 0)),
          pl.BlockSpec((1, d), lambda i: (0, 0)),
          pl.BlockSpec((1, d), lambda i: (0, 0)),
          pl.BlockSpec((1, d), lambda i: (0, 0)),
      ],
      out_specs=pl.BlockSpec((bn, d), lambda i: (i, 0)),
      out_shape=jax.ShapeDtypeStruct((n, d), jnp.float32),
  )(h, agg2, u1a, u1b, ub1, u2, ub2, ln_g, ln_b)


def _mean_rows(h, bn):
  n, d = h.shape
  nbn = pl.cdiv(n, bn)

  def body(h_r, out):
    i = pl.program_id(0)

    @pl.when(i == 0)
    def _():
      out[...] = jnp.zeros_like(out)

    rows = i * bn + lax.broadcasted_iota(jnp.int32, (bn, 1), 0)
    x = jnp.where(rows < n, h_r[...], 0.0)
    out[...] += jnp.sum(x, axis=0, keepdims=True) * (1.0 / n)

  return pl.pallas_call(
      body,
      grid=(nbn,),
      in_specs=[pl.BlockSpec((bn, d), lambda i: (i, 0))],
      out_specs=pl.BlockSpec((1, d), lambda i: (0, 0)),
      out_shape=jax.ShapeDtypeStruct((1, d), jnp.float32),
  )(h)


def kernel(z, pos, edge_index, atom_embed, layers):
  n, d = pos.shape[0], atom_embed.shape[1]
  e = edge_index.shape[1]
  row = edge_index[0].astype(jnp.int32)
  col = edge_index[1].astype(jnp.int32)
  npad = ((n + 2047) // 2048) * 2048

  g_rows = ((2 * e + 16383) // 16384) * 16384
  gpad = jnp.arange(g_rows - 2 * e, dtype=jnp.int32) % n
  idxcat = jnp.concatenate([row, col, gpad]).reshape(-1, 4, _L)

  e_pad = ((e + 8191) // 8192) * 8192
  tpad = n + jnp.arange(e_pad - e, dtype=jnp.int32) % (npad - n)
  row3 = jnp.concatenate([row, tpad]).reshape(-1, 2, _L)

  posf = pos.astype(jnp.float32)
  z_rows = ((n + 16383) // 16384) * 16384
  z3 = jnp.pad(z.astype(jnp.int32), (0, z_rows - n)).reshape(-1, 4, _L)
  h = _gather_rows(atom_embed.astype(jnp.float32), z3, z_rows)[:n]
  d2 = _edge_d2(posf[:, 0], posf[:, 1], posf[:, 2], idxcat, e, 512)
  d2 = jnp.pad(d2, (0, e_pad - e)).reshape(e_pad, 1)

  for lp in layers:
    w1 = lp['msg_w1']
    hcat = _gather_rows(h, idxcat, g_rows)
    m = _edge_mlp(hcat, d2,
                  w1[:d], w1[d:2 * d], w1[2 * d:].reshape(1, d),
                  lp['msg_b1'].reshape(1, d), lp['msg_w2'],
                  lp['msg_b2'].reshape(1, d), e_pad, e // 640, 640)
    agg2 = _scatter_add(m, row3, npad)
    u1 = lp['upd_w1']
    h = _node_update(h, agg2, u1[:d], u1[d:],
                     lp['upd_b1'].reshape(1, d), lp['upd_w2'],
                     lp['upd_b2'].reshape(1, d),
                     lp['ln_g'].reshape(1, d), lp['ln_b'].reshape(1, d), 512)

  return _mean_rows(h, 512)


# speedup vs baseline: 1.6989x; 1.0954x over previous
import functools

import jax
import jax.numpy as jnp
from jax import lax
from jax.experimental import pallas as pl
from jax.experimental.pallas import tpu as pltpu
from jax.experimental.pallas import tpu_sc as plsc

_NC = 2
_NS = 16
_NW = _NC * _NS
_L = 128


def _gather_rows(table, idx3, n_rows):
  _, d = table.shape
  quads = n_rows // 512
  mesh = plsc.VectorSubcoreMesh(core_axis_name="c", subcore_axis_name="s")

  @functools.partial(
      pl.kernel,
      mesh=mesh,
      out_type=jax.ShapeDtypeStruct((n_rows, d), table.dtype),
      scratch_types=[
          pltpu.VMEM((4, _L), jnp.int32),
          pltpu.VMEM((512, d), table.dtype),
          pltpu.SemaphoreType.DMA,
      ],
  )
  def gk(table_hbm, idx_hbm, out_hbm, idx_v, rows_v, sem):
    wid = lax.axis_index("s") * _NC + lax.axis_index("c")
    nt = (quads - wid + _NW - 1) // _NW

    def body(u, carry):
      qi = wid + u * _NW
      base = qi * 512
      pltpu.sync_copy(idx_hbm.at[qi], idx_v)
      cps = [
          pltpu.async_copy(table_hbm.at[idx_v.at[j]],
                           rows_v.at[pl.ds(j * _L, _L)], sem)
          for j in range(4)
      ]
      for cp in cps:
        cp.wait()
      pltpu.sync_copy(rows_v, out_hbm.at[pl.ds(base, 512)])
      return carry

    lax.fori_loop(0, nt, body, 0)

  return gk(table, idx3)


def _scatter_add(msg, idx3, n_pad):
  n_msg, d = msg.shape
  pairs = n_msg // 256
  rps = n_pad // _NS
  mesh = plsc.VectorSubcoreMesh(core_axis_name="c", subcore_axis_name="s")

  @functools.partial(
      pl.kernel,
      mesh=mesh,
      out_type=jax.ShapeDtypeStruct((_NC, n_pad, d), msg.dtype),
      scratch_types=[
          pltpu.VMEM((2, _L), jnp.int32),
          pltpu.VMEM((_L, d), msg.dtype),
          pltpu.VMEM((_L, d), msg.dtype),
          pltpu.VMEM_SHARED((n_pad, d), msg.dtype),
          pltpu.SemaphoreType.DMA,
          pltpu.SemaphoreType.DMA,
          pltpu.SemaphoreType.DMA,
          pltpu.SemaphoreType.DMA,
      ],
  )
  def sk(m_hbm, idx_hbm, zeros_hbm, out_hbm, idx_v, buf_a, buf_b, agg_sh,
         sem_la, sem_lb, sem_sa, sem_sb):
    cid = lax.axis_index("c")
    sid = lax.axis_index("s")
    wid = sid * _NC + cid
    pltpu.sync_copy(zeros_hbm.at[pl.ds(sid * rps, rps)],
                    agg_sh.at[pl.ds(sid * rps, rps)])
    plsc.subcore_barrier()
    nt = (pairs - wid + _NW - 1) // _NW

    def body(u, carry):
      pi = wid + u * _NW
      base = pi * 256
      pltpu.sync_copy(idx_hbm.at[pi], idx_v)
      la = pltpu.async_copy(m_hbm.at[pl.ds(base, _L)], buf_a, sem_la)
      lb = pltpu.async_copy(m_hbm.at[pl.ds(base + _L, _L)], buf_b, sem_lb)
      la.wait()
      sa = pltpu.async_copy(buf_a, agg_sh.at[idx_v.at[0]], sem_sa, add=True)
      lb.wait()
      sb = pltpu.async_copy(buf_b, agg_sh.at[idx_v.at[1]], sem_sb, add=True)
      sa.wait()
      sb.wait()
      return carry

    lax.fori_loop(0, nt, body, 0)
    plsc.subcore_barrier()
    pltpu.sync_copy(agg_sh.at[pl.ds(sid * rps, rps)],
                    out_hbm.at[cid, pl.ds(sid * rps, rps)])

  return sk(msg, idx3, jnp.zeros((n_pad, d), msg.dtype))


def _edge_d2(posx, posy, posz, idx3, n_edges, chunk):
  n = posx.shape[0]
  k = chunk // _L
  n_chunks = n_edges // chunk
  mesh = plsc.VectorSubcoreMesh(core_axis_name="c", subcore_axis_name="s")

  @functools.partial(
      pl.kernel,
      mesh=mesh,
      out_type=jax.ShapeDtypeStruct((n_edges,), jnp.float32),
      compiler_params=pltpu.CompilerParams(needs_layout_passes=False),
      scratch_types=[
          pltpu.VMEM((n,), jnp.float32),
          pltpu.VMEM((n,), jnp.float32),
          pltpu.VMEM((n,), jnp.float32),
          pltpu.VMEM((k, _L), jnp.int32),
          pltpu.VMEM((k, _L), jnp.int32),
          pltpu.VMEM((chunk,), jnp.float32),
      ],
  )
  def dk(px_hbm, py_hbm, pz_hbm, idx_hbm, out_hbm, px, py, pz, ir_v, ic_v,
         d2_v):
    wid = lax.axis_index("s") * _NC + lax.axis_index("c")
    pltpu.sync_copy(px_hbm, px)
    pltpu.sync_copy(py_hbm, py)
    pltpu.sync_copy(pz_hbm, pz)
    nt = (n_chunks - wid + _NW - 1) // _NW

    def body(t, carry):
      g = wid + t * _NW
      pltpu.sync_copy(idx_hbm.at[g], ir_v)
      pltpu.sync_copy(idx_hbm.at[g + n_chunks], ic_v)
      for j in range(k):
        def vec(v, c2):
          ir = ir_v[j, pl.ds(v * 16, 16)]
          ic = ic_v[j, pl.ds(v * 16, 16)]
          dx = plsc.load_gather(px, [ir]) - plsc.load_gather(px, [ic])
          dy = plsc.load_gather(py, [ir]) - plsc.load_gather(py, [ic])
          dz = plsc.load_gather(pz, [ir]) - plsc.load_gather(pz, [ic])
          d2_v[pl.ds(j * _L + v * 16, 16)] = dx * dx + dy * dy + dz * dz
          return c2

        lax.fori_loop(0, _L // 16, vec, 0)
      pltpu.sync_copy(d2_v, out_hbm.at[pl.ds(g * chunk, chunk)])
      return carry

    lax.fori_loop(0, nt, body, 0)

  return dk(posx, posy, posz, idx3)


def _edge_mlp(hcat, d2, w1a, w1b, w1d, b1, w2, b2, n_out, col_off, be):
  d = hcat.shape[1]
  nbe = n_out // be

  def body(hr, hc, d2_r, w1a_r, w1b_r, w1d_r, b1_r, w2_r, b2_r, out):
    dist = jnp.sqrt(d2_r[...] + 1e-8)
    t = (jnp.dot(hr[...], w1a_r[...], preferred_element_type=jnp.float32)
         + jnp.dot(hc[...], w1b_r[...], preferred_element_type=jnp.float32)
         + dist * w1d_r[...] + b1_r[...])
    t = t * jax.nn.sigmoid(t)
    out[...] = jnp.dot(t, w2_r[...],
                       preferred_element_type=jnp.float32) + b2_r[...]

  return pl.pallas_call(
      body,
      grid=(nbe,),
      in_specs=[
          pl.BlockSpec((be, d), lambda i: (i, 0)),
          pl.BlockSpec((be, d), lambda i: (i + col_off, 0)),
          pl.BlockSpec((be, 1), lambda i: (i, 0)),
          pl.BlockSpec((d, d), lambda i: (0, 0)),
          pl.BlockSpec((d, d), lambda i: (0, 0)),
          pl.BlockSpec((1, d), lambda i: (0, 0)),
          pl.BlockSpec((1, d), lambda i: (0, 0)),
          pl.BlockSpec((d, d), lambda i: (0, 0)),
          pl.BlockSpec((1, d), lambda i: (0, 0)),
      ],
      out_specs=pl.BlockSpec((be, d), lambda i: (i, 0)),
      out_shape=jax.ShapeDtypeStruct((n_out, d), jnp.float32),
  )(hcat, hcat, d2, w1a, w1b, w1d, b1, w2, b2)


def _node_update(h, agg2, u1a, u1b, ub1, u2, ub2, ln_g, ln_b, bn):
  n, d = h.shape
  nbn = pl.cdiv(n, bn)

  def body(h_r, a_r, u1a_r, u1b_r, ub1_r, u2_r, ub2_r, g_r, b_r, out):
    hv = h_r[...]
    a = a_r[0] + a_r[1]
    t = (jnp.dot(hv, u1a_r[...], preferred_element_type=jnp.float32)
         + jnp.dot(a, u1b_r[...], preferred_element_type=jnp.float32)
         + ub1_r[...])
    t = t * jax.nn.sigmoid(t)
    u = jnp.dot(t, u2_r[...], preferred_element_type=jnp.float32) + ub2_r[...]
    r = hv + u
    mu = jnp.mean(r, axis=-1, keepdims=True)
    var = jnp.mean((r - mu) ** 2, axis=-1, keepdims=True)
    out[...] = (r - mu) / jnp.sqrt(var + 1e-5) * g_r[...] + b_r[...]

  return pl.pallas_call(
      body,
      grid=(nbn,),
      in_specs=[
          pl.BlockSpec((bn, d), lambda i: (i, 0)),
          pl.BlockSpec((2, bn, d), lambda i: (0, i, 0)),
          pl.BlockSpec((d, d), lambda i: (0, 0)),
          pl.BlockSpec((d, d), lambda i: (0, 0)),
          pl.BlockSpec((1, d), lambda i: (0, 0)),
          pl.BlockSpec((d, d), lambda i: (0, 0)),
          pl.BlockSpec((1, d), lambda i: (0, 0)),
          pl.BlockSpec((1, d), lambda i: (0, 0)),
          pl.BlockSpec((1, d), lambda i: (0, 0)),
      ],
      out_specs=pl.BlockSpec((bn, d), lambda i: (i, 0)),
      out_shape=jax.ShapeDtypeStruct((n, d), jnp.float32),
  )(h, agg2, u1a, u1b, ub1, u2, ub2, ln_g, ln_b)


def _mean_rows(h, bn):
  n, d = h.shape
  nbn = pl.cdiv(n, bn)

  def body(h_r, out):
    i = pl.program_id(0)

    @pl.when(i == 0)
    def _():
      out[...] = jnp.zeros_like(out)

    rows = i * bn + lax.broadcasted_iota(jnp.int32, (bn, 1), 0)
    x = jnp.where(rows < n, h_r[...], 0.0)
    out[...] += jnp.sum(x, axis=0, keepdims=True) * (1.0 / n)

  return pl.pallas_call(
      body,
      grid=(nbn,),
      in_specs=[pl.BlockSpec((bn, d), lambda i: (i, 0))],
      out_specs=pl.BlockSpec((1, d), lambda i: (0, 0)),
      out_shape=jax.ShapeDtypeStruct((1, d), jnp.float32),
  )(h)


def kernel(z, pos, edge_index, atom_embed, layers):
  n, d = pos.shape[0], atom_embed.shape[1]
  e = edge_index.shape[1]
  row = edge_index[0].astype(jnp.int32)
  col = edge_index[1].astype(jnp.int32)
  npad = ((n + 2047) // 2048) * 2048

  idxcat = jnp.concatenate([row, col]).reshape(-1, 4, _L)
  row3 = row.reshape(-1, 2, _L)

  posf = pos.astype(jnp.float32)
  z_rows = ((n + 511) // 512) * 512
  zpad = jnp.arange(z_rows - n, dtype=jnp.int32) % atom_embed.shape[0]
  z3 = jnp.concatenate([z.astype(jnp.int32), zpad]).reshape(-1, 4, _L)
  h = _gather_rows(atom_embed.astype(jnp.float32), z3, z_rows)[:n]
  d2 = _edge_d2(posf[:, 0], posf[:, 1], posf[:, 2], idxcat, e, 512)
  d2 = d2.reshape(e, 1)

  for lp in layers:
    w1 = lp['msg_w1']
    hcat = _gather_rows(h, idxcat, 2 * e)
    m = _edge_mlp(hcat, d2,
                  w1[:d], w1[d:2 * d], w1[2 * d:].reshape(1, d),
                  lp['msg_b1'].reshape(1, d), lp['msg_w2'],
                  lp['msg_b2'].reshape(1, d), e, e // 640, 640)
    agg2 = _scatter_add(m, row3, npad)
    u1 = lp['upd_w1']
    h = _node_update(h, agg2, u1[:d], u1[d:],
                     lp['upd_b1'].reshape(1, d), lp['upd_w2'],
                     lp['upd_b2'].reshape(1, d),
                     lp['ln_g'].reshape(1, d), lp['ln_b'].reshape(1, d), 512)

  return _mean_rows(h, 512)


# speedup vs baseline: 1.7189x; 1.0117x over previous
import functools

import jax
import jax.numpy as jnp
from jax import lax
from jax.experimental import pallas as pl
from jax.experimental.pallas import tpu as pltpu
from jax.experimental.pallas import tpu_sc as plsc

_NC = 2
_NS = 16
_NW = _NC * _NS
_L = 128


def _gather_rows(table, idx3, n_rows, k=4):
  _, d = table.shape
  chunk = k * _L
  n_chunks = n_rows // chunk
  mesh = plsc.VectorSubcoreMesh(core_axis_name="c", subcore_axis_name="s")

  @functools.partial(
      pl.kernel,
      mesh=mesh,
      out_type=jax.ShapeDtypeStruct((n_rows, d), table.dtype),
      scratch_types=[
          pltpu.VMEM((k, _L), jnp.int32),
          pltpu.VMEM((chunk, d), table.dtype),
          pltpu.SemaphoreType.DMA,
      ],
  )
  def gk(table_hbm, idx_hbm, out_hbm, idx_v, rows_v, sem):
    wid = lax.axis_index("s") * _NC + lax.axis_index("c")
    nt = (n_chunks - wid + _NW - 1) // _NW

    def body(u, carry):
      qi = wid + u * _NW
      base = qi * chunk
      pltpu.sync_copy(idx_hbm.at[qi], idx_v)
      cps = [
          pltpu.async_copy(table_hbm.at[idx_v.at[j]],
                           rows_v.at[pl.ds(j * _L, _L)], sem)
          for j in range(k)
      ]
      for cp in cps:
        cp.wait()
      pltpu.sync_copy(rows_v, out_hbm.at[pl.ds(base, chunk)])
      return carry

    lax.fori_loop(0, nt, body, 0)

  return gk(table, idx3)


def _scatter_add(msg, idx3, n_pad):
  n_msg, d = msg.shape
  pairs = n_msg // 256
  rps = n_pad // _NS
  mesh = plsc.VectorSubcoreMesh(core_axis_name="c", subcore_axis_name="s")

  @functools.partial(
      pl.kernel,
      mesh=mesh,
      out_type=jax.ShapeDtypeStruct((_NC, n_pad, d), msg.dtype),
      scratch_types=[
          pltpu.VMEM((2, _L), jnp.int32),
          pltpu.VMEM((_L, d), msg.dtype),
          pltpu.VMEM((_L, d), msg.dtype),
          pltpu.VMEM_SHARED((n_pad, d), msg.dtype),
          pltpu.SemaphoreType.DMA,
          pltpu.SemaphoreType.DMA,
          pltpu.SemaphoreType.DMA,
          pltpu.SemaphoreType.DMA,
      ],
  )
  def sk(m_hbm, idx_hbm, zeros_hbm, out_hbm, idx_v, buf_a, buf_b, agg_sh,
         sem_la, sem_lb, sem_sa, sem_sb):
    cid = lax.axis_index("c")
    sid = lax.axis_index("s")
    wid = sid * _NC + cid
    pltpu.sync_copy(zeros_hbm.at[pl.ds(sid * rps, rps)],
                    agg_sh.at[pl.ds(sid * rps, rps)])
    plsc.subcore_barrier()
    nt = (pairs - wid + _NW - 1) // _NW

    def body(u, carry):
      pi = wid + u * _NW
      base = pi * 256
      pltpu.sync_copy(idx_hbm.at[pi], idx_v)
      la = pltpu.async_copy(m_hbm.at[pl.ds(base, _L)], buf_a, sem_la)
      lb = pltpu.async_copy(m_hbm.at[pl.ds(base + _L, _L)], buf_b, sem_lb)
      la.wait()
      sa = pltpu.async_copy(buf_a, agg_sh.at[idx_v.at[0]], sem_sa, add=True)
      lb.wait()
      sb = pltpu.async_copy(buf_b, agg_sh.at[idx_v.at[1]], sem_sb, add=True)
      sa.wait()
      sb.wait()
      return carry

    lax.fori_loop(0, nt, body, 0)
    plsc.subcore_barrier()
    pltpu.sync_copy(agg_sh.at[pl.ds(sid * rps, rps)],
                    out_hbm.at[cid, pl.ds(sid * rps, rps)])

  return sk(msg, idx3, jnp.zeros((n_pad, d), msg.dtype))


def _edge_d2(posx, posy, posz, idx3, n_edges, chunk):
  n = posx.shape[0]
  k = chunk // _L
  n_chunks = n_edges // chunk
  mesh = plsc.VectorSubcoreMesh(core_axis_name="c", subcore_axis_name="s")

  @functools.partial(
      pl.kernel,
      mesh=mesh,
      out_type=jax.ShapeDtypeStruct((n_edges,), jnp.float32),
      compiler_params=pltpu.CompilerParams(needs_layout_passes=False),
      scratch_types=[
          pltpu.VMEM((n,), jnp.float32),
          pltpu.VMEM((n,), jnp.float32),
          pltpu.VMEM((n,), jnp.float32),
          pltpu.VMEM((k, _L), jnp.int32),
          pltpu.VMEM((k, _L), jnp.int32),
          pltpu.VMEM((chunk,), jnp.float32),
      ],
  )
  def dk(px_hbm, py_hbm, pz_hbm, idx_hbm, out_hbm, px, py, pz, ir_v, ic_v,
         d2_v):
    wid = lax.axis_index("s") * _NC + lax.axis_index("c")
    pltpu.sync_copy(px_hbm, px)
    pltpu.sync_copy(py_hbm, py)
    pltpu.sync_copy(pz_hbm, pz)
    nt = (n_chunks - wid + _NW - 1) // _NW

    def body(t, carry):
      g = wid + t * _NW
      pltpu.sync_copy(idx_hbm.at[g], ir_v)
      pltpu.sync_copy(idx_hbm.at[g + n_chunks], ic_v)
      for j in range(k):
        def vec(v, c2):
          ir = ir_v[j, pl.ds(v * 16, 16)]
          ic = ic_v[j, pl.ds(v * 16, 16)]
          dx = plsc.load_gather(px, [ir]) - plsc.load_gather(px, [ic])
          dy = plsc.load_gather(py, [ir]) - plsc.load_gather(py, [ic])
          dz = plsc.load_gather(pz, [ir]) - plsc.load_gather(pz, [ic])
          d2_v[pl.ds(j * _L + v * 16, 16)] = dx * dx + dy * dy + dz * dz
          return c2

        lax.fori_loop(0, _L // 16, vec, 0)
      pltpu.sync_copy(d2_v, out_hbm.at[pl.ds(g * chunk, chunk)])
      return carry

    lax.fori_loop(0, nt, body, 0)

  return dk(posx, posy, posz, idx3)


def _edge_mlp(hcat, d2, w1a, w1b, w1d, b1, w2, b2, n_out, col_off, be):
  d = hcat.shape[1]
  nbe = n_out // be

  def body(hr, hc, d2_r, w1a_r, w1b_r, w1d_r, b1_r, w2_r, b2_r, out):
    bf = jnp.bfloat16
    dist = jnp.sqrt(d2_r[...] + 1e-8)
    t = (jnp.dot(hr[...].astype(bf), w1a_r[...].astype(bf),
                 preferred_element_type=jnp.float32)
         + jnp.dot(hc[...].astype(bf), w1b_r[...].astype(bf),
                   preferred_element_type=jnp.float32)
         + dist * w1d_r[...] + b1_r[...])
    t = t * jax.nn.sigmoid(t)
    out[...] = jnp.dot(t.astype(bf), w2_r[...].astype(bf),
                       preferred_element_type=jnp.float32) + b2_r[...]

  return pl.pallas_call(
      body,
      grid=(nbe,),
      in_specs=[
          pl.BlockSpec((be, d), lambda i: (i, 0)),
          pl.BlockSpec((be, d), lambda i: (i + col_off, 0)),
          pl.BlockSpec((be, 1), lambda i: (i, 0)),
          pl.BlockSpec((d, d), lambda i: (0, 0)),
          pl.BlockSpec((d, d), lambda i: (0, 0)),
          pl.BlockSpec((1, d), lambda i: (0, 0)),
          pl.BlockSpec((1, d), lambda i: (0, 0)),
          pl.BlockSpec((d, d), lambda i: (0, 0)),
          pl.BlockSpec((1, d), lambda i: (0, 0)),
      ],
      out_specs=pl.BlockSpec((be, d), lambda i: (i, 0)),
      out_shape=jax.ShapeDtypeStruct((n_out, d), jnp.float32),
  )(hcat, hcat, d2, w1a, w1b, w1d, b1, w2, b2)


def _node_update(h, agg2, u1a, u1b, ub1, u2, ub2, ln_g, ln_b, bn):
  n, d = h.shape
  nbn = pl.cdiv(n, bn)

  def body(h_r, a_r, u1a_r, u1b_r, ub1_r, u2_r, ub2_r, g_r, b_r, out):
    hv = h_r[...]
    a = a_r[0] + a_r[1]
    t = (jnp.dot(hv, u1a_r[...], preferred_element_type=jnp.float32)
         + jnp.dot(a, u1b_r[...], preferred_element_type=jnp.float32)
         + ub1_r[...])
    t = t * jax.nn.sigmoid(t)
    u = jnp.dot(t, u2_r[...], preferred_element_type=jnp.float32) + ub2_r[...]
    r = hv + u
    mu = jnp.mean(r, axis=-1, keepdims=True)
    var = jnp.mean((r - mu) ** 2, axis=-1, keepdims=True)
    out[...] = (r - mu) / jnp.sqrt(var + 1e-5) * g_r[...] + b_r[...]

  return pl.pallas_call(
      body,
      grid=(nbn,),
      in_specs=[
          pl.BlockSpec((bn, d), lambda i: (i, 0)),
          pl.BlockSpec((2, bn, d), lambda i: (0, i, 0)),
          pl.BlockSpec((d, d), lambda i: (0, 0)),
          pl.BlockSpec((d, d), lambda i: (0, 0)),
          pl.BlockSpec((1, d), lambda i: (0, 0)),
          pl.BlockSpec((d, d), lambda i: (0, 0)),
          pl.BlockSpec((1, d), lambda i: (0, 0)),
          pl.BlockSpec((1, d), lambda i: (0, 0)),
          pl.BlockSpec((1, d), lambda i: (0, 0)),
      ],
      out_specs=pl.BlockSpec((bn, d), lambda i: (i, 0)),
      out_shape=jax.ShapeDtypeStruct((n, d), jnp.float32),
  )(h, agg2, u1a, u1b, ub1, u2, ub2, ln_g, ln_b)


def _mean_rows(h, bn):
  n, d = h.shape
  nbn = pl.cdiv(n, bn)

  def body(h_r, out):
    i = pl.program_id(0)

    @pl.when(i == 0)
    def _():
      out[...] = jnp.zeros_like(out)

    rows = i * bn + lax.broadcasted_iota(jnp.int32, (bn, 1), 0)
    x = jnp.where(rows < n, h_r[...], 0.0)
    out[...] += jnp.sum(x, axis=0, keepdims=True) * (1.0 / n)

  return pl.pallas_call(
      body,
      grid=(nbn,),
      in_specs=[pl.BlockSpec((bn, d), lambda i: (i, 0))],
      out_specs=pl.BlockSpec((1, d), lambda i: (0, 0)),
      out_shape=jax.ShapeDtypeStruct((1, d), jnp.float32),
  )(h)


def kernel(z, pos, edge_index, atom_embed, layers):
  n, d = pos.shape[0], atom_embed.shape[1]
  e = edge_index.shape[1]
  row = edge_index[0].astype(jnp.int32)
  col = edge_index[1].astype(jnp.int32)
  npad = ((n + 2047) // 2048) * 2048

  idxflat = jnp.concatenate([row, col])
  idxcat = idxflat.reshape(-1, 5, _L)
  idxcat_d2 = idxflat.reshape(-1, 4, _L)
  row3 = row.reshape(-1, 2, _L)

  posf = pos.astype(jnp.float32)
  z_rows = ((n + 511) // 512) * 512
  zpad = jnp.arange(z_rows - n, dtype=jnp.int32) % atom_embed.shape[0]
  z3 = jnp.concatenate([z.astype(jnp.int32), zpad]).reshape(-1, 4, _L)
  h = _gather_rows(atom_embed.astype(jnp.float32), z3, z_rows)
  h = h[:n]
  d2 = _edge_d2(posf[:, 0], posf[:, 1], posf[:, 2], idxcat_d2, e, 512)
  d2 = d2.reshape(e, 1)

  for lp in layers:
    w1 = lp['msg_w1']
    hcat = _gather_rows(h, idxcat, 2 * e, k=5)
    m = _edge_mlp(hcat, d2,
                  w1[:d], w1[d:2 * d], w1[2 * d:].reshape(1, d),
                  lp['msg_b1'].reshape(1, d), lp['msg_w2'],
                  lp['msg_b2'].reshape(1, d), e, e // 640, 640)
    agg2 = _scatter_add(m, row3, npad)
    u1 = lp['upd_w1']
    h = _node_update(h, agg2, u1[:d], u1[d:],
                     lp['upd_b1'].reshape(1, d), lp['upd_w2'],
                     lp['upd_b2'].reshape(1, d),
                     lp['ln_g'].reshape(1, d), lp['ln_b'].reshape(1, d), 512)

  return _mean_rows(h, 512)


# speedup vs baseline: 1.7384x; 1.0114x over previous
import functools

import jax
import jax.numpy as jnp
from jax import lax
from jax.experimental import pallas as pl
from jax.experimental.pallas import tpu as pltpu
from jax.experimental.pallas import tpu_sc as plsc

_NC = 2
_NS = 16
_NW = _NC * _NS
_L = 128


def _gather_rows(table, idx3, n_rows, k=4):
  _, d = table.shape
  chunk = k * _L
  n_chunks = n_rows // chunk
  mesh = plsc.VectorSubcoreMesh(core_axis_name="c", subcore_axis_name="s")

  @functools.partial(
      pl.kernel,
      mesh=mesh,
      out_type=jax.ShapeDtypeStruct((n_rows, d), table.dtype),
      scratch_types=[
          pltpu.VMEM((k, _L), jnp.int32),
          pltpu.VMEM((chunk, d), table.dtype),
          pltpu.SemaphoreType.DMA,
      ],
  )
  def gk(table_hbm, idx_hbm, out_hbm, idx_v, rows_v, sem):
    wid = lax.axis_index("s") * _NC + lax.axis_index("c")
    nt = (n_chunks - wid + _NW - 1) // _NW

    def body(u, carry):
      qi = wid + u * _NW
      base = qi * chunk
      pltpu.sync_copy(idx_hbm.at[qi], idx_v)
      cps = [
          pltpu.async_copy(table_hbm.at[idx_v.at[j]],
                           rows_v.at[pl.ds(j * _L, _L)], sem)
          for j in range(k)
      ]
      for cp in cps:
        cp.wait()
      pltpu.sync_copy(rows_v, out_hbm.at[pl.ds(base, chunk)])
      return carry

    lax.fori_loop(0, nt, body, 0)

  return gk(table, idx3)


def _scatter_add(msg, idx3, n_pad):
  n_msg, d = msg.shape
  pairs = n_msg // 256
  rps = n_pad // _NS
  mesh = plsc.VectorSubcoreMesh(core_axis_name="c", subcore_axis_name="s")

  @functools.partial(
      pl.kernel,
      mesh=mesh,
      out_type=jax.ShapeDtypeStruct((_NC, n_pad, d), msg.dtype),
      scratch_types=[
          pltpu.VMEM((2, _L), jnp.int32),
          pltpu.VMEM((_L, d), msg.dtype),
          pltpu.VMEM((_L, d), msg.dtype),
          pltpu.VMEM_SHARED((n_pad, d), msg.dtype),
          pltpu.SemaphoreType.DMA,
          pltpu.SemaphoreType.DMA,
          pltpu.SemaphoreType.DMA,
          pltpu.SemaphoreType.DMA,
      ],
  )
  def sk(m_hbm, idx_hbm, zeros_hbm, out_hbm, idx_v, buf_a, buf_b, agg_sh,
         sem_la, sem_lb, sem_sa, sem_sb):
    cid = lax.axis_index("c")
    sid = lax.axis_index("s")
    wid = sid * _NC + cid
    pltpu.sync_copy(zeros_hbm.at[pl.ds(sid * rps, rps)],
                    agg_sh.at[pl.ds(sid * rps, rps)])
    plsc.subcore_barrier()
    nt = (pairs - wid + _NW - 1) // _NW

    def body(u, carry):
      pi = wid + u * _NW
      base = pi * 256
      pltpu.sync_copy(idx_hbm.at[pi], idx_v)
      la = pltpu.async_copy(m_hbm.at[pl.ds(base, _L)], buf_a, sem_la)
      lb = pltpu.async_copy(m_hbm.at[pl.ds(base + _L, _L)], buf_b, sem_lb)
      la.wait()
      sa = pltpu.async_copy(buf_a, agg_sh.at[idx_v.at[0]], sem_sa, add=True)
      lb.wait()
      sb = pltpu.async_copy(buf_b, agg_sh.at[idx_v.at[1]], sem_sb, add=True)
      sa.wait()
      sb.wait()
      return carry

    lax.fori_loop(0, nt, body, 0)
    plsc.subcore_barrier()
    pltpu.sync_copy(agg_sh.at[pl.ds(sid * rps, rps)],
                    out_hbm.at[cid, pl.ds(sid * rps, rps)])

  return sk(msg, idx3, jnp.zeros((n_pad, d), msg.dtype))


def _edge_d2(posx, posy, posz, idx3, n_edges, chunk):
  n = posx.shape[0]
  k = chunk // _L
  n_chunks = n_edges // chunk
  mesh = plsc.VectorSubcoreMesh(core_axis_name="c", subcore_axis_name="s")

  @functools.partial(
      pl.kernel,
      mesh=mesh,
      out_type=jax.ShapeDtypeStruct((n_edges,), jnp.float32),
      compiler_params=pltpu.CompilerParams(needs_layout_passes=False),
      scratch_types=[
          pltpu.VMEM((n,), jnp.float32),
          pltpu.VMEM((n,), jnp.float32),
          pltpu.VMEM((n,), jnp.float32),
          pltpu.VMEM((k, _L), jnp.int32),
          pltpu.VMEM((k, _L), jnp.int32),
          pltpu.VMEM((chunk,), jnp.float32),
      ],
  )
  def dk(px_hbm, py_hbm, pz_hbm, idx_hbm, out_hbm, px, py, pz, ir_v, ic_v,
         d2_v):
    wid = lax.axis_index("s") * _NC + lax.axis_index("c")
    pltpu.sync_copy(px_hbm, px)
    pltpu.sync_copy(py_hbm, py)
    pltpu.sync_copy(pz_hbm, pz)
    nt = (n_chunks - wid + _NW - 1) // _NW

    def body(t, carry):
      g = wid + t * _NW
      pltpu.sync_copy(idx_hbm.at[g], ir_v)
      pltpu.sync_copy(idx_hbm.at[g + n_chunks], ic_v)
      for j in range(k):
        def vec(v, c2):
          ir = ir_v[j, pl.ds(v * 16, 16)]
          ic = ic_v[j, pl.ds(v * 16, 16)]
          dx = plsc.load_gather(px, [ir]) - plsc.load_gather(px, [ic])
          dy = plsc.load_gather(py, [ir]) - plsc.load_gather(py, [ic])
          dz = plsc.load_gather(pz, [ir]) - plsc.load_gather(pz, [ic])
          d2_v[pl.ds(j * _L + v * 16, 16)] = dx * dx + dy * dy + dz * dz
          return c2

        lax.fori_loop(0, _L // 16, vec, 0)
      pltpu.sync_copy(d2_v, out_hbm.at[pl.ds(g * chunk, chunk)])
      return carry

    lax.fori_loop(0, nt, body, 0)

  return dk(posx, posy, posz, idx3)


def _edge_mlp(hcat, d2, w1a, w1b, w1d, b1, w2, b2, n_out, col_off, be):
  d = hcat.shape[1]
  nbe = n_out // be

  def body(hr, hc, d2_r, w1a_r, w1b_r, w1d_r, b1_r, w2_r, b2_r, out):
    bf = jnp.bfloat16
    dist = jnp.sqrt(d2_r[...] + 1e-8)
    t = (jnp.dot(hr[...].astype(bf), w1a_r[...].astype(bf),
                 preferred_element_type=jnp.float32)
         + jnp.dot(hc[...].astype(bf), w1b_r[...].astype(bf),
                   preferred_element_type=jnp.float32)
         + dist * w1d_r[...] + b1_r[...])
    t = t * jax.nn.sigmoid(t)
    out[...] = jnp.dot(t.astype(bf), w2_r[...].astype(bf),
                       preferred_element_type=jnp.float32) + b2_r[...]

  return pl.pallas_call(
      body,
      grid=(nbe,),
      in_specs=[
          pl.BlockSpec((be, d), lambda i: (i, 0)),
          pl.BlockSpec((be, d), lambda i: (i + col_off, 0)),
          pl.BlockSpec((be, 1), lambda i: (i, 0)),
          pl.BlockSpec((d, d), lambda i: (0, 0)),
          pl.BlockSpec((d, d), lambda i: (0, 0)),
          pl.BlockSpec((1, d), lambda i: (0, 0)),
          pl.BlockSpec((1, d), lambda i: (0, 0)),
          pl.BlockSpec((d, d), lambda i: (0, 0)),
          pl.BlockSpec((1, d), lambda i: (0, 0)),
      ],
      out_specs=pl.BlockSpec((be, d), lambda i: (i, 0)),
      out_shape=jax.ShapeDtypeStruct((n_out, d), jnp.float32),
  )(hcat, hcat, d2, w1a, w1b, w1d, b1, w2, b2)


def _node_update(h, agg2, u1a, u1b, ub1, u2, ub2, ln_g, ln_b, bn):
  n, d = h.shape
  nbn = pl.cdiv(n, bn)

  def body(h_r, a_r, u1a_r, u1b_r, ub1_r, u2_r, ub2_r, g_r, b_r, out):
    hv = h_r[...]
    a = a_r[0] + a_r[1]
    t = (jnp.dot(hv, u1a_r[...], preferred_element_type=jnp.float32)
         + jnp.dot(a, u1b_r[...], preferred_element_type=jnp.float32)
         + ub1_r[...])
    t = t * jax.nn.sigmoid(t)
    u = jnp.dot(t, u2_r[...], preferred_element_type=jnp.float32) + ub2_r[...]
    r = hv + u
    mu = jnp.mean(r, axis=-1, keepdims=True)
    var = jnp.mean((r - mu) ** 2, axis=-1, keepdims=True)
    out[...] = (r - mu) / jnp.sqrt(var + 1e-5) * g_r[...] + b_r[...]

  return pl.pallas_call(
      body,
      grid=(nbn,),
      in_specs=[
          pl.BlockSpec((bn, d), lambda i: (i, 0)),
          pl.BlockSpec((2, bn, d), lambda i: (0, i, 0)),
          pl.BlockSpec((d, d), lambda i: (0, 0)),
          pl.BlockSpec((d, d), lambda i: (0, 0)),
          pl.BlockSpec((1, d), lambda i: (0, 0)),
          pl.BlockSpec((d, d), lambda i: (0, 0)),
          pl.BlockSpec((1, d), lambda i: (0, 0)),
          pl.BlockSpec((1, d), lambda i: (0, 0)),
          pl.BlockSpec((1, d), lambda i: (0, 0)),
      ],
      out_specs=pl.BlockSpec((bn, d), lambda i: (i, 0)),
      out_shape=jax.ShapeDtypeStruct((n, d), jnp.float32),
  )(h, agg2, u1a, u1b, ub1, u2, ub2, ln_g, ln_b)


def _mean_rows(h, bn):
  n, d = h.shape
  nbn = pl.cdiv(n, bn)

  def body(h_r, out):
    i = pl.program_id(0)

    @pl.when(i == 0)
    def _():
      out[...] = jnp.zeros_like(out)

    rows = i * bn + lax.broadcasted_iota(jnp.int32, (bn, 1), 0)
    x = jnp.where(rows < n, h_r[...], 0.0)
    out[...] += jnp.sum(x, axis=0, keepdims=True) * (1.0 / n)

  return pl.pallas_call(
      body,
      grid=(nbn,),
      in_specs=[pl.BlockSpec((bn, d), lambda i: (i, 0))],
      out_specs=pl.BlockSpec((1, d), lambda i: (0, 0)),
      out_shape=jax.ShapeDtypeStruct((1, d), jnp.float32),
  )(h)


def kernel(z, pos, edge_index, atom_embed, layers):
  n, d = pos.shape[0], atom_embed.shape[1]
  e = edge_index.shape[1]
  row = edge_index[0].astype(jnp.int32)
  col = edge_index[1].astype(jnp.int32)
  npad = ((n + 2047) // 2048) * 2048

  idxflat = jnp.concatenate([row, col])
  g_rows = ((2 * e + 895) // 896) * 896
  gpad = jnp.arange(g_rows - 2 * e, dtype=jnp.int32) % n
  idxcat = jnp.concatenate([idxflat, gpad]).reshape(-1, 7, _L)
  idxcat_d2 = idxflat.reshape(-1, 4, _L)
  row3 = row.reshape(-1, 2, _L)

  posf = pos.astype(jnp.float32)
  z_rows = ((n + 511) // 512) * 512
  zpad = jnp.arange(z_rows - n, dtype=jnp.int32) % atom_embed.shape[0]
  z3 = jnp.concatenate([z.astype(jnp.int32), zpad]).reshape(-1, 4, _L)
  h = _gather_rows(atom_embed.astype(jnp.float32), z3, z_rows)
  h = h[:n]
  d2 = _edge_d2(posf[:, 0], posf[:, 1], posf[:, 2], idxcat_d2, e, 512)
  d2 = d2.reshape(e, 1)

  for lp in layers:
    w1 = lp['msg_w1']
    hcat = _gather_rows(h, idxcat, g_rows, k=7)
    m = _edge_mlp(hcat, d2,
                  w1[:d], w1[d:2 * d], w1[2 * d:].reshape(1, d),
                  lp['msg_b1'].reshape(1, d), lp['msg_w2'],
                  lp['msg_b2'].reshape(1, d), e, e // 640, 640)
    agg2 = _scatter_add(m, row3, npad)
    u1 = lp['upd_w1']
    h = _node_update(h, agg2, u1[:d], u1[d:],
                     lp['upd_b1'].reshape(1, d), lp['upd_w2'],
                     lp['upd_b2'].reshape(1, d),
                     lp['ln_g'].reshape(1, d), lp['ln_b'].reshape(1, d), 512)

  return _mean_rows(h, 512)


# speedup vs baseline: 1.9235x; 1.1065x over previous
import functools

import jax
import jax.numpy as jnp
from jax import lax
from jax.experimental import pallas as pl
from jax.experimental.pallas import tpu as pltpu
from jax.experimental.pallas import tpu_sc as plsc

_NC = 2
_NS = 16
_NW = _NC * _NS
_L = 128


def _gather_rows(table, idx3, n_rows, k=4):
  _, d = table.shape
  chunk = k * _L
  n_chunks = n_rows // chunk
  mesh = plsc.VectorSubcoreMesh(core_axis_name="c", subcore_axis_name="s")

  @functools.partial(
      pl.kernel,
      mesh=mesh,
      out_type=jax.ShapeDtypeStruct((n_rows, d), table.dtype),
      scratch_types=[
          pltpu.VMEM((k, _L), jnp.int32),
          pltpu.VMEM((chunk, d), table.dtype),
          pltpu.SemaphoreType.DMA,
      ],
  )
  def gk(table_hbm, idx_hbm, out_hbm, idx_v, rows_v, sem):
    wid = lax.axis_index("s") * _NC + lax.axis_index("c")
    nt = (n_chunks - wid + _NW - 1) // _NW

    def body(u, carry):
      qi = wid + u * _NW
      base = qi * chunk
      pltpu.sync_copy(idx_hbm.at[qi], idx_v)
      cps = [
          pltpu.async_copy(table_hbm.at[idx_v.at[j]],
                           rows_v.at[pl.ds(j * _L, _L)], sem)
          for j in range(k)
      ]
      for cp in cps:
        cp.wait()
      pltpu.sync_copy(rows_v, out_hbm.at[pl.ds(base, chunk)])
      return carry

    lax.fori_loop(0, nt, body, 0)

  return gk(table, idx3)


def _scatter_add(msg, idx3, n_pad):
  n_msg, d = msg.shape
  pairs = n_msg // 256
  rps = n_pad // _NS
  mesh = plsc.VectorSubcoreMesh(core_axis_name="c", subcore_axis_name="s")

  @functools.partial(
      pl.kernel,
      mesh=mesh,
      out_type=jax.ShapeDtypeStruct((_NC, n_pad, d), msg.dtype),
      scratch_types=[
          pltpu.VMEM((2, _L), jnp.int32),
          pltpu.VMEM((_L, d), msg.dtype),
          pltpu.VMEM((_L, d), msg.dtype),
          pltpu.VMEM_SHARED((n_pad, d), msg.dtype),
          pltpu.SemaphoreType.DMA,
          pltpu.SemaphoreType.DMA,
          pltpu.SemaphoreType.DMA,
          pltpu.SemaphoreType.DMA,
      ],
  )
  def sk(m_hbm, idx_hbm, zeros_hbm, out_hbm, idx_v, buf_a, buf_b, agg_sh,
         sem_la, sem_lb, sem_sa, sem_sb):
    cid = lax.axis_index("c")
    sid = lax.axis_index("s")
    wid = sid * _NC + cid
    pltpu.sync_copy(zeros_hbm.at[pl.ds(sid * rps, rps)],
                    agg_sh.at[pl.ds(sid * rps, rps)])
    plsc.subcore_barrier()
    nt = (pairs - wid + _NW - 1) // _NW

    def body(u, carry):
      pi = wid + u * _NW
      base = pi * 256
      pltpu.sync_copy(idx_hbm.at[pi], idx_v)
      la = pltpu.async_copy(m_hbm.at[pl.ds(base, _L)], buf_a, sem_la)
      lb = pltpu.async_copy(m_hbm.at[pl.ds(base + _L, _L)], buf_b, sem_lb)
      la.wait()
      sa = pltpu.async_copy(buf_a, agg_sh.at[idx_v.at[0]], sem_sa, add=True)
      lb.wait()
      sb = pltpu.async_copy(buf_b, agg_sh.at[idx_v.at[1]], sem_sb, add=True)
      sa.wait()
      sb.wait()
      return carry

    lax.fori_loop(0, nt, body, 0)
    plsc.subcore_barrier()
    pltpu.sync_copy(agg_sh.at[pl.ds(sid * rps, rps)],
                    out_hbm.at[cid, pl.ds(sid * rps, rps)])

  return sk(msg, idx3, jnp.zeros((n_pad, d), msg.dtype))


def _gather_sum(ta, tb, idxr3, idxc3, n_out):
  _, d = ta.shape
  chunk = 7 * _L
  n_chunks = n_out // chunk
  mesh = plsc.VectorSubcoreMesh(core_axis_name="c", subcore_axis_name="s")

  @functools.partial(
      pl.kernel,
      mesh=mesh,
      out_type=jax.ShapeDtypeStruct((n_out, d), ta.dtype),
      scratch_types=[
          pltpu.VMEM((7, _L), jnp.int32),
          pltpu.VMEM((7, _L), jnp.int32),
          pltpu.VMEM((chunk, d), ta.dtype),
          pltpu.SemaphoreType.DMA,
      ],
  )
  def gk(ta_hbm, tb_hbm, idxr_hbm, idxc_hbm, out_hbm, idxr_v, idxc_v,
         rows_v, sem):
    wid = lax.axis_index("s") * _NC + lax.axis_index("c")
    nt = (n_chunks - wid + _NW - 1) // _NW

    def body(u, carry):
      qi = wid + u * _NW
      base = qi * chunk
      pltpu.sync_copy(idxr_hbm.at[qi], idxr_v)
      pltpu.sync_copy(idxc_hbm.at[qi], idxc_v)
      cps = [
          pltpu.async_copy(ta_hbm.at[idxr_v.at[j]],
                           rows_v.at[pl.ds(j * _L, _L)], sem)
          for j in range(7)
      ]
      for cp in cps:
        cp.wait()
      cps = [
          pltpu.async_copy(tb_hbm.at[idxc_v.at[j]],
                           rows_v.at[pl.ds(j * _L, _L)], sem, add=True)
          for j in range(7)
      ]
      for cp in cps:
        cp.wait()
      pltpu.sync_copy(rows_v, out_hbm.at[pl.ds(base, chunk)])
      return carry

    lax.fori_loop(0, nt, body, 0)

  return gk(ta, tb, idxr3, idxc3)


def _pre_ab(h, w1a, w1b, bn):
  n, d = h.shape
  nbn = pl.cdiv(n, bn)

  def body(h_r, w1a_r, w1b_r, a_out, b_out):
    hv = h_r[...]
    a_out[...] = jnp.dot(hv, w1a_r[...], preferred_element_type=jnp.float32)
    b_out[...] = jnp.dot(hv, w1b_r[...], preferred_element_type=jnp.float32)

  return pl.pallas_call(
      body,
      grid=(nbn,),
      in_specs=[
          pl.BlockSpec((bn, d), lambda i: (i, 0)),
          pl.BlockSpec((d, d), lambda i: (0, 0)),
          pl.BlockSpec((d, d), lambda i: (0, 0)),
      ],
      out_specs=[
          pl.BlockSpec((bn, d), lambda i: (i, 0)),
          pl.BlockSpec((bn, d), lambda i: (i, 0)),
      ],
      out_shape=[
          jax.ShapeDtypeStruct((n, d), jnp.float32),
          jax.ShapeDtypeStruct((n, d), jnp.float32),
      ],
  )(h, w1a, w1b)


def _edge_d2(posx, posy, posz, idx3, n_edges, chunk):
  n = posx.shape[0]
  k = chunk // _L
  n_chunks = n_edges // chunk
  mesh = plsc.VectorSubcoreMesh(core_axis_name="c", subcore_axis_name="s")

  @functools.partial(
      pl.kernel,
      mesh=mesh,
      out_type=jax.ShapeDtypeStruct((n_edges,), jnp.float32),
      compiler_params=pltpu.CompilerParams(needs_layout_passes=False),
      scratch_types=[
          pltpu.VMEM((n,), jnp.float32),
          pltpu.VMEM((n,), jnp.float32),
          pltpu.VMEM((n,), jnp.float32),
          pltpu.VMEM((k, _L), jnp.int32),
          pltpu.VMEM((k, _L), jnp.int32),
          pltpu.VMEM((chunk,), jnp.float32),
      ],
  )
  def dk(px_hbm, py_hbm, pz_hbm, idx_hbm, out_hbm, px, py, pz, ir_v, ic_v,
         d2_v):
    wid = lax.axis_index("s") * _NC + lax.axis_index("c")
    pltpu.sync_copy(px_hbm, px)
    pltpu.sync_copy(py_hbm, py)
    pltpu.sync_copy(pz_hbm, pz)
    nt = (n_chunks - wid + _NW - 1) // _NW

    def body(t, carry):
      g = wid + t * _NW
      pltpu.sync_copy(idx_hbm.at[g], ir_v)
      pltpu.sync_copy(idx_hbm.at[g + n_chunks], ic_v)
      for j in range(k):
        def vec(v, c2):
          ir = ir_v[j, pl.ds(v * 16, 16)]
          ic = ic_v[j, pl.ds(v * 16, 16)]
          dx = plsc.load_gather(px, [ir]) - plsc.load_gather(px, [ic])
          dy = plsc.load_gather(py, [ir]) - plsc.load_gather(py, [ic])
          dz = plsc.load_gather(pz, [ir]) - plsc.load_gather(pz, [ic])
          d2_v[pl.ds(j * _L + v * 16, 16)] = dx * dx + dy * dy + dz * dz
          return c2

        lax.fori_loop(0, _L // 16, vec, 0)
      pltpu.sync_copy(d2_v, out_hbm.at[pl.ds(g * chunk, chunk)])
      return carry

    lax.fori_loop(0, nt, body, 0)

  return dk(posx, posy, posz, idx3)


def _edge_mlp(tpre, d2, w1d, b1, w2, b2, n_edges, be):
  d = w2.shape[0]
  nbe = n_edges // be

  def body(tp, d2_r, w1d_r, b1_r, w2_r, b2_r, out):
    bf = jnp.bfloat16
    dist = jnp.sqrt(d2_r[...] + 1e-8)
    t = tp[...] + dist * w1d_r[...] + b1_r[...]
    t = t * jax.nn.sigmoid(t)
    out[...] = jnp.dot(t.astype(bf), w2_r[...].astype(bf),
                       preferred_element_type=jnp.float32) + b2_r[...]

  return pl.pallas_call(
      body,
      grid=(nbe,),
      in_specs=[
          pl.BlockSpec((be, d), lambda i: (i, 0)),
          pl.BlockSpec((be, 1), lambda i: (i, 0)),
          pl.BlockSpec((1, d), lambda i: (0, 0)),
          pl.BlockSpec((1, d), lambda i: (0, 0)),
          pl.BlockSpec((d, d), lambda i: (0, 0)),
          pl.BlockSpec((1, d), lambda i: (0, 0)),
      ],
      out_specs=pl.BlockSpec((be, d), lambda i: (i, 0)),
      out_shape=jax.ShapeDtypeStruct((n_edges, d), jnp.float32),
  )(tpre, d2, w1d, b1, w2, b2)


def _node_update(h, agg2, u1a, u1b, ub1, u2, ub2, ln_g, ln_b, bn):
  n, d = h.shape
  nbn = pl.cdiv(n, bn)

  def body(h_r, a_r, u1a_r, u1b_r, ub1_r, u2_r, ub2_r, g_r, b_r, out):
    hv = h_r[...]
    a = a_r[0] + a_r[1]
    t = (jnp.dot(hv, u1a_r[...], preferred_element_type=jnp.float32)
         + jnp.dot(a, u1b_r[...], preferred_element_type=jnp.float32)
         + ub1_r[...])
    t = t * jax.nn.sigmoid(t)
    u = jnp.dot(t, u2_r[...], preferred_element_type=jnp.float32) + ub2_r[...]
    r = hv + u
    mu = jnp.mean(r, axis=-1, keepdims=True)
    var = jnp.mean((r - mu) ** 2, axis=-1, keepdims=True)
    out[...] = (r - mu) / jnp.sqrt(var + 1e-5) * g_r[...] + b_r[...]

  return pl.pallas_call(
      body,
      grid=(nbn,),
      in_specs=[
          pl.BlockSpec((bn, d), lambda i: (i, 0)),
          pl.BlockSpec((2, bn, d), lambda i: (0, i, 0)),
          pl.BlockSpec((d, d), lambda i: (0, 0)),
          pl.BlockSpec((d, d), lambda i: (0, 0)),
          pl.BlockSpec((1, d), lambda i: (0, 0)),
          pl.BlockSpec((d, d), lambda i: (0, 0)),
          pl.BlockSpec((1, d), lambda i: (0, 0)),
          pl.BlockSpec((1, d), lambda i: (0, 0)),
          pl.BlockSpec((1, d), lambda i: (0, 0)),
      ],
      out_specs=pl.BlockSpec((bn, d), lambda i: (i, 0)),
      out_shape=jax.ShapeDtypeStruct((n, d), jnp.float32),
  )(h, agg2, u1a, u1b, ub1, u2, ub2, ln_g, ln_b)


def _mean_rows(h, bn):
  n, d = h.shape
  nbn = pl.cdiv(n, bn)

  def body(h_r, out):
    i = pl.program_id(0)

    @pl.when(i == 0)
    def _():
      out[...] = jnp.zeros_like(out)

    rows = i * bn + lax.broadcasted_iota(jnp.int32, (bn, 1), 0)
    x = jnp.where(rows < n, h_r[...], 0.0)
    out[...] += jnp.sum(x, axis=0, keepdims=True) * (1.0 / n)

  return pl.pallas_call(
      body,
      grid=(nbn,),
      in_specs=[pl.BlockSpec((bn, d), lambda i: (i, 0))],
      out_specs=pl.BlockSpec((1, d), lambda i: (0, 0)),
      out_shape=jax.ShapeDtypeStruct((1, d), jnp.float32),
  )(h)


def kernel(z, pos, edge_index, atom_embed, layers):
  n, d = pos.shape[0], atom_embed.shape[1]
  e = edge_index.shape[1]
  row = edge_index[0].astype(jnp.int32)
  col = edge_index[1].astype(jnp.int32)
  npad = ((n + 2047) // 2048) * 2048

  e_pre = ((e + 895) // 896) * 896
  ipad = jnp.arange(e_pre - e, dtype=jnp.int32) % n
  idxr3 = jnp.concatenate([row, ipad]).reshape(-1, 7, _L)
  idxc3 = jnp.concatenate([col, ipad]).reshape(-1, 7, _L)
  idxcat_d2 = jnp.concatenate([row, col]).reshape(-1, 4, _L)
  row3 = row.reshape(-1, 2, _L)

  posf = pos.astype(jnp.float32)
  z_rows = ((n + 511) // 512) * 512
  zpad = jnp.arange(z_rows - n, dtype=jnp.int32) % atom_embed.shape[0]
  z3 = jnp.concatenate([z.astype(jnp.int32), zpad]).reshape(-1, 4, _L)
  h = _gather_rows(atom_embed.astype(jnp.float32), z3, z_rows)
  h = h[:n]
  d2 = _edge_d2(posf[:, 0], posf[:, 1], posf[:, 2], idxcat_d2, e, 512)
  d2 = d2.reshape(e, 1)

  for lp in layers:
    w1 = lp['msg_w1']
    ta, tb = _pre_ab(h, w1[:d], w1[d:2 * d], 512)
    tpre = _gather_sum(ta, tb, idxr3, idxc3, e_pre)
    m = _edge_mlp(tpre, d2, w1[2 * d:].reshape(1, d),
                  lp['msg_b1'].reshape(1, d), lp['msg_w2'],
                  lp['msg_b2'].reshape(1, d), e, 640)
    agg2 = _scatter_add(m, row3, npad)
    u1 = lp['upd_w1']
    h = _node_update(h, agg2, u1[:d], u1[d:],
                     lp['upd_b1'].reshape(1, d), lp['upd_w2'],
                     lp['upd_b2'].reshape(1, d),
                     lp['ln_g'].reshape(1, d), lp['ln_b'].reshape(1, d), 512)

  return _mean_rows(h, 512)


# speedup vs baseline: 1.9423x; 1.0098x over previous
import functools

import jax
import jax.numpy as jnp
from jax import lax
from jax.experimental import pallas as pl
from jax.experimental.pallas import tpu as pltpu
from jax.experimental.pallas import tpu_sc as plsc

_NC = 2
_NS = 16
_NW = _NC * _NS
_L = 128


def _gather_rows(table, idx3, n_rows, k=4):
  _, d = table.shape
  chunk = k * _L
  n_chunks = n_rows // chunk
  mesh = plsc.VectorSubcoreMesh(core_axis_name="c", subcore_axis_name="s")

  @functools.partial(
      pl.kernel,
      mesh=mesh,
      out_type=jax.ShapeDtypeStruct((n_rows, d), table.dtype),
      scratch_types=[
          pltpu.VMEM((k, _L), jnp.int32),
          pltpu.VMEM((chunk, d), table.dtype),
          pltpu.SemaphoreType.DMA,
      ],
  )
  def gk(table_hbm, idx_hbm, out_hbm, idx_v, rows_v, sem):
    wid = lax.axis_index("s") * _NC + lax.axis_index("c")
    nt = (n_chunks - wid + _NW - 1) // _NW

    def body(u, carry):
      qi = wid + u * _NW
      base = qi * chunk
      pltpu.sync_copy(idx_hbm.at[qi], idx_v)
      cps = [
          pltpu.async_copy(table_hbm.at[idx_v.at[j]],
                           rows_v.at[pl.ds(j * _L, _L)], sem)
          for j in range(k)
      ]
      for cp in cps:
        cp.wait()
      pltpu.sync_copy(rows_v, out_hbm.at[pl.ds(base, chunk)])
      return carry

    lax.fori_loop(0, nt, body, 0)

  return gk(table, idx3)


def _scatter_add(msg, idx3, n_pad):
  n_msg, d = msg.shape
  pairs = n_msg // 256
  rps = n_pad // _NS
  mesh = plsc.VectorSubcoreMesh(core_axis_name="c", subcore_axis_name="s")

  @functools.partial(
      pl.kernel,
      mesh=mesh,
      out_type=jax.ShapeDtypeStruct((_NC, n_pad, d), msg.dtype),
      scratch_types=[
          pltpu.VMEM((2, _L), jnp.int32),
          pltpu.VMEM((_L, d), msg.dtype),
          pltpu.VMEM((_L, d), msg.dtype),
          pltpu.VMEM_SHARED((n_pad, d), msg.dtype),
          pltpu.SemaphoreType.DMA,
          pltpu.SemaphoreType.DMA,
          pltpu.SemaphoreType.DMA,
          pltpu.SemaphoreType.DMA,
      ],
  )
  def sk(m_hbm, idx_hbm, zeros_hbm, out_hbm, idx_v, buf_a, buf_b, agg_sh,
         sem_la, sem_lb, sem_sa, sem_sb):
    cid = lax.axis_index("c")
    sid = lax.axis_index("s")
    wid = sid * _NC + cid
    pltpu.sync_copy(zeros_hbm.at[pl.ds(sid * rps, rps)],
                    agg_sh.at[pl.ds(sid * rps, rps)])
    plsc.subcore_barrier()
    nt = (pairs - wid + _NW - 1) // _NW

    def body(u, carry):
      pi = wid + u * _NW
      base = pi * 256
      pltpu.sync_copy(idx_hbm.at[pi], idx_v)
      la = pltpu.async_copy(m_hbm.at[pl.ds(base, _L)], buf_a, sem_la)
      lb = pltpu.async_copy(m_hbm.at[pl.ds(base + _L, _L)], buf_b, sem_lb)
      la.wait()
      sa = pltpu.async_copy(buf_a, agg_sh.at[idx_v.at[0]], sem_sa, add=True)
      lb.wait()
      sb = pltpu.async_copy(buf_b, agg_sh.at[idx_v.at[1]], sem_sb, add=True)
      sa.wait()
      sb.wait()
      return carry

    lax.fori_loop(0, nt, body, 0)
    plsc.subcore_barrier()
    pltpu.sync_copy(agg_sh.at[pl.ds(sid * rps, rps)],
                    out_hbm.at[cid, pl.ds(sid * rps, rps)])

  return sk(msg, idx3, jnp.zeros((n_pad, d), msg.dtype))


def _gather_sum(ta, tb, idxr3, idxc3, n_out):
  _, d = ta.shape
  chunk = 7 * _L
  n_chunks = n_out // chunk
  mesh = plsc.VectorSubcoreMesh(core_axis_name="c", subcore_axis_name="s")

  @functools.partial(
      pl.kernel,
      mesh=mesh,
      out_type=jax.ShapeDtypeStruct((n_out, d), ta.dtype),
      scratch_types=[
          pltpu.VMEM((7, _L), jnp.int32),
          pltpu.VMEM((7, _L), jnp.int32),
          pltpu.VMEM((chunk, d), ta.dtype),
          pltpu.SemaphoreType.DMA,
      ],
  )
  def gk(ta_hbm, tb_hbm, idxr_hbm, idxc_hbm, out_hbm, idxr_v, idxc_v,
         rows_v, sem):
    wid = lax.axis_index("s") * _NC + lax.axis_index("c")
    nt = (n_chunks - wid + _NW - 1) // _NW

    def body(u, carry):
      qi = wid + u * _NW
      base = qi * chunk
      pltpu.sync_copy(idxr_hbm.at[qi], idxr_v)
      pltpu.sync_copy(idxc_hbm.at[qi], idxc_v)
      cps = [
          pltpu.async_copy(ta_hbm.at[idxr_v.at[j]],
                           rows_v.at[pl.ds(j * _L, _L)], sem)
          for j in range(7)
      ]
      for cp in cps:
        cp.wait()
      cps = [
          pltpu.async_copy(tb_hbm.at[idxc_v.at[j]],
                           rows_v.at[pl.ds(j * _L, _L)], sem, add=True)
          for j in range(7)
      ]
      for cp in cps:
        cp.wait()
      pltpu.sync_copy(rows_v, out_hbm.at[pl.ds(base, chunk)])
      return carry

    lax.fori_loop(0, nt, body, 0)

  return gk(ta, tb, idxr3, idxc3)


def _pre_ab(h, w1a, w1b, bn):
  n, d = h.shape
  nbn = pl.cdiv(n, bn)

  def body(h_r, w1a_r, w1b_r, a_out, b_out):
    hv = h_r[...]
    a_out[...] = jnp.dot(hv, w1a_r[...], preferred_element_type=jnp.float32)
    b_out[...] = jnp.dot(hv, w1b_r[...], preferred_element_type=jnp.float32)

  return pl.pallas_call(
      body,
      grid=(nbn,),
      in_specs=[
          pl.BlockSpec((bn, d), lambda i: (i, 0)),
          pl.BlockSpec((d, d), lambda i: (0, 0)),
          pl.BlockSpec((d, d), lambda i: (0, 0)),
      ],
      out_specs=[
          pl.BlockSpec((bn, d), lambda i: (i, 0)),
          pl.BlockSpec((bn, d), lambda i: (i, 0)),
      ],
      out_shape=[
          jax.ShapeDtypeStruct((n, d), jnp.float32),
          jax.ShapeDtypeStruct((n, d), jnp.float32),
      ],
  )(h, w1a, w1b)


def _edge_d2(posx, posy, posz, idx3, n_edges, chunk):
  n = posx.shape[0]
  k = chunk // _L
  n_chunks = n_edges // chunk
  mesh = plsc.VectorSubcoreMesh(core_axis_name="c", subcore_axis_name="s")

  @functools.partial(
      pl.kernel,
      mesh=mesh,
      out_type=jax.ShapeDtypeStruct((n_edges,), jnp.float32),
      compiler_params=pltpu.CompilerParams(needs_layout_passes=False),
      scratch_types=[
          pltpu.VMEM((n,), jnp.float32),
          pltpu.VMEM((n,), jnp.float32),
          pltpu.VMEM((n,), jnp.float32),
          pltpu.VMEM((k, _L), jnp.int32),
          pltpu.VMEM((k, _L), jnp.int32),
          pltpu.VMEM((chunk,), jnp.float32),
      ],
  )
  def dk(px_hbm, py_hbm, pz_hbm, idx_hbm, out_hbm, px, py, pz, ir_v, ic_v,
         d2_v):
    wid = lax.axis_index("s") * _NC + lax.axis_index("c")
    pltpu.sync_copy(px_hbm, px)
    pltpu.sync_copy(py_hbm, py)
    pltpu.sync_copy(pz_hbm, pz)
    nt = (n_chunks - wid + _NW - 1) // _NW

    def body(t, carry):
      g = wid + t * _NW
      pltpu.sync_copy(idx_hbm.at[g], ir_v)
      pltpu.sync_copy(idx_hbm.at[g + n_chunks], ic_v)
      for j in range(k):
        def vec(v, c2):
          ir = ir_v[j, pl.ds(v * 16, 16)]
          ic = ic_v[j, pl.ds(v * 16, 16)]
          dx = plsc.load_gather(px, [ir]) - plsc.load_gather(px, [ic])
          dy = plsc.load_gather(py, [ir]) - plsc.load_gather(py, [ic])
          dz = plsc.load_gather(pz, [ir]) - plsc.load_gather(pz, [ic])
          d2_v[pl.ds(j * _L + v * 16, 16)] = dx * dx + dy * dy + dz * dz
          return c2

        lax.fori_loop(0, _L // 16, vec, 0)
      pltpu.sync_copy(d2_v, out_hbm.at[pl.ds(g * chunk, chunk)])
      return carry

    lax.fori_loop(0, nt, body, 0)

  return dk(posx, posy, posz, idx3)


def _edge_mlp(tpre, d2, w1d, b1, w2, b2, n_edges, be):
  d = w2.shape[0]
  nbe = n_edges // be

  def body(tp, d2_r, w1d_r, b1_r, w2_r, b2_r, out):
    bf = jnp.bfloat16
    dist = jnp.sqrt(d2_r[...] + 1e-8)
    t = tp[...] + dist * w1d_r[...] + b1_r[...]
    t = t * jax.nn.sigmoid(t)
    out[...] = jnp.dot(t.astype(bf), w2_r[...].astype(bf),
                       preferred_element_type=jnp.float32) + b2_r[...]

  return pl.pallas_call(
      body,
      grid=(nbe,),
      in_specs=[
          pl.BlockSpec((be, d), lambda i: (i, 0)),
          pl.BlockSpec((be, 1), lambda i: (i, 0)),
          pl.BlockSpec((1, d), lambda i: (0, 0)),
          pl.BlockSpec((1, d), lambda i: (0, 0)),
          pl.BlockSpec((d, d), lambda i: (0, 0)),
          pl.BlockSpec((1, d), lambda i: (0, 0)),
      ],
      out_specs=pl.BlockSpec((be, d), lambda i: (i, 0)),
      out_shape=jax.ShapeDtypeStruct((n_edges, d), jnp.float32),
  )(tpre, d2, w1d, b1, w2, b2)


def _node_update(h, agg2, u1a, u1b, ub1, u2, ub2, ln_g, ln_b, bn,
                 w1a_n=None, w1b_n=None):
  n, d = h.shape
  nbn = pl.cdiv(n, bn)
  with_ab = w1a_n is not None

  def body(h_r, a_r, u1a_r, u1b_r, ub1_r, u2_r, ub2_r, g_r, b_r, *rest):
    hv = h_r[...]
    a = a_r[0] + a_r[1]
    t = (jnp.dot(hv, u1a_r[...], preferred_element_type=jnp.float32)
         + jnp.dot(a, u1b_r[...], preferred_element_type=jnp.float32)
         + ub1_r[...])
    t = t * jax.nn.sigmoid(t)
    u = jnp.dot(t, u2_r[...], preferred_element_type=jnp.float32) + ub2_r[...]
    r = hv + u
    mu = jnp.mean(r, axis=-1, keepdims=True)
    var = jnp.mean((r - mu) ** 2, axis=-1, keepdims=True)
    hn = (r - mu) / jnp.sqrt(var + 1e-5) * g_r[...] + b_r[...]
    if with_ab:
      w1a_r, w1b_r, out, a_out, b_out = rest
      a_out[...] = jnp.dot(hn, w1a_r[...], preferred_element_type=jnp.float32)
      b_out[...] = jnp.dot(hn, w1b_r[...], preferred_element_type=jnp.float32)
    else:
      out, = rest
    out[...] = hn

  full = pl.BlockSpec((d, d), lambda i: (0, 0))
  vec = pl.BlockSpec((1, d), lambda i: (0, 0))
  rows = pl.BlockSpec((bn, d), lambda i: (i, 0))
  in_specs = [
      rows,
      pl.BlockSpec((2, bn, d), lambda i: (0, i, 0)),
      full, full, vec, full, vec, vec, vec,
  ]
  args = [h, agg2, u1a, u1b, ub1, u2, ub2, ln_g, ln_b]
  out_specs = rows
  out_shape = jax.ShapeDtypeStruct((n, d), jnp.float32)
  if with_ab:
    in_specs += [full, full]
    args += [w1a_n, w1b_n]
    out_specs = [rows, rows, rows]
    out_shape = [out_shape, out_shape, out_shape]
  return pl.pallas_call(
      body,
      grid=(nbn,),
      in_specs=in_specs,
      out_specs=out_specs,
      out_shape=out_shape,
  )(*args)


def _mean_rows(h, bn):
  n, d = h.shape
  nbn = pl.cdiv(n, bn)

  def body(h_r, out):
    i = pl.program_id(0)

    @pl.when(i == 0)
    def _():
      out[...] = jnp.zeros_like(out)

    rows = i * bn + lax.broadcasted_iota(jnp.int32, (bn, 1), 0)
    x = jnp.where(rows < n, h_r[...], 0.0)
    out[...] += jnp.sum(x, axis=0, keepdims=True) * (1.0 / n)

  return pl.pallas_call(
      body,
      grid=(nbn,),
      in_specs=[pl.BlockSpec((bn, d), lambda i: (i, 0))],
      out_specs=pl.BlockSpec((1, d), lambda i: (0, 0)),
      out_shape=jax.ShapeDtypeStruct((1, d), jnp.float32),
  )(h)


def kernel(z, pos, edge_index, atom_embed, layers):
  n, d = pos.shape[0], atom_embed.shape[1]
  e = edge_index.shape[1]
  row = edge_index[0].astype(jnp.int32)
  col = edge_index[1].astype(jnp.int32)
  npad = ((n + 2047) // 2048) * 2048

  e_pre = ((e + 895) // 896) * 896
  ipad = jnp.arange(e_pre - e, dtype=jnp.int32) % n
  idxr3 = jnp.concatenate([row, ipad]).reshape(-1, 7, _L)
  idxc3 = jnp.concatenate([col, ipad]).reshape(-1, 7, _L)
  idxcat_d2 = jnp.concatenate([row, col]).reshape(-1, 4, _L)
  row3 = row.reshape(-1, 2, _L)

  posf = pos.astype(jnp.float32)
  z_rows = ((n + 511) // 512) * 512
  zpad = jnp.arange(z_rows - n, dtype=jnp.int32) % atom_embed.shape[0]
  z3 = jnp.concatenate([z.astype(jnp.int32), zpad]).reshape(-1, 4, _L)
  h = _gather_rows(atom_embed.astype(jnp.float32), z3, z_rows)
  h = h[:n]
  d2 = _edge_d2(posf[:, 0], posf[:, 1], posf[:, 2], idxcat_d2, e, 512)
  d2 = d2.reshape(e, 1)

  w1_0 = layers[0]['msg_w1']
  ta, tb = _pre_ab(h, w1_0[:d], w1_0[d:2 * d], 512)
  for li, lp in enumerate(layers):
    w1 = lp['msg_w1']
    tpre = _gather_sum(ta, tb, idxr3, idxc3, e_pre)
    m = _edge_mlp(tpre, d2, w1[2 * d:].reshape(1, d),
                  lp['msg_b1'].reshape(1, d), lp['msg_w2'],
                  lp['msg_b2'].reshape(1, d), e, 640)
    agg2 = _scatter_add(m, row3, npad)
    u1 = lp['upd_w1']
    last = li + 1 == len(layers)
    w1n = None if last else layers[li + 1]['msg_w1']
    res = _node_update(h, agg2, u1[:d], u1[d:],
                       lp['upd_b1'].reshape(1, d), lp['upd_w2'],
                       lp['upd_b2'].reshape(1, d),
                       lp['ln_g'].reshape(1, d), lp['ln_b'].reshape(1, d),
                       512,
                       None if last else w1n[:d],
                       None if last else w1n[d:2 * d])
    if last:
      h = res
    else:
      h, ta, tb = res

  return _mean_rows(h, 512)
